# baseline (speedup 1.0000x reference)
import functools
import math

import jax
import jax.numpy as jnp
from jax import lax
from jax.experimental import pallas as pl
from jax.experimental.pallas import tpu as pltpu
from jax.experimental.pallas import tpu_sc as plsc

N = 10240
NP = 10000
D = 256
G = 64
EP = 163840
NB = 40
BLK = 256

f32 = jnp.float32
i32 = jnp.int32

def _sc_mesh():
    return plsc.VectorSubcoreMesh(core_axis_name="c", subcore_axis_name="s",
                                  num_cores=2, num_subcores=16)


@functools.lru_cache(maxsize=None)
def _sc_spmm_build():
    return pl.kernel(
        _sc_spmm_body,
        out_type=jax.ShapeDtypeStruct((2, N, 128), f32),
        mesh=_sc_mesh(),
        scratch_types=[
            pltpu.VMEM((128,), i32),
            pltpu.VMEM((128,), i32),
            pltpu.VMEM((128,), i32),
            pltpu.VMEM((128, 128), f32),
            pltpu.VMEM((128, 128), f32),
            pltpu.VMEM_SHARED((N, 128), f32),
            pltpu.SemaphoreType.DMA,
            pltpu.SemaphoreType.DMA,
        ],
    )


def _sc_spmm(g_flat, src_spmm, dst_spmm, zrows):
    return _sc_spmm_build()(g_flat, src_spmm, dst_spmm, zrows)


def _sc_spmm_body(g_hbm, srcm_hbm, dstm_hbm, zrows_hbm, out_hbm,
                  sidx0, sidx1, didx, rows0, rows1, acc_sh, sem0, sem1):
    c = lax.axis_index("c")
    s = lax.axis_index("s")
    sidx = (sidx0, sidx1)
    rows = (rows0, rows1)
    sems = (sem0, sem1)
    pltpu.sync_copy(zrows_hbm, acc_sh.at[pl.ds(s * 640, 640)])
    plsc.subcore_barrier()

    def prefetch(k, bi):
        pltpu.sync_copy(srcm_hbm.at[s, k], sidx[bi])

        def adj(j, _):
            sidx[bi][pl.ds(j * 16, 16)] = sidx[bi][pl.ds(j * 16, 16)] + c * N
            return 0
        lax.fori_loop(0, 8, adj, 0, unroll=True)
        pltpu.async_copy(g_hbm.at[sidx[bi]], rows[bi], sems[bi])

    def consume(k, bi):
        pltpu.make_async_copy(g_hbm.at[sidx[bi]], rows[bi], sems[bi]).wait()
        pltpu.sync_copy(dstm_hbm.at[s, k], didx)
        pltpu.sync_copy(rows[bi], acc_sh.at[didx], add=True)

    prefetch(0, 0)

    def body(i, _):
        prefetch(2 * i + 1, 1)
        consume(2 * i, 0)

        @pl.when(i < 39)
        def _():
            prefetch(2 * i + 2, 0)
        consume(2 * i + 1, 1)
        return 0

    lax.fori_loop(0, 40, body, 0)
    plsc.subcore_barrier()
    pltpu.sync_copy(acc_sh.at[pl.ds(s * 640, 640)],
                    out_hbm.at[c].at[pl.ds(s * 640, 640)])


@functools.lru_cache(maxsize=None)
def _sc_deg_build():
    return pl.kernel(
        _sc_deg_body,
        out_type=jax.ShapeDtypeStruct((2, N), f32),
        mesh=_sc_mesh(),
        scratch_types=[
            pltpu.VMEM((128,), i32),
            pltpu.VMEM((128,), i32),
            pltpu.VMEM((128,), i32),
            pltpu.VMEM((128,), f32),
            pltpu.VMEM((128,), f32),
            pltpu.VMEM_SHARED((N,), f32),
            pltpu.SemaphoreType.DMA,
            pltpu.SemaphoreType.DMA,
        ],
    )


def _sc_deg(w, src_deg, dst_deg, zvec):
    return _sc_deg_build()(w, src_deg, dst_deg, zvec)


def _sc_deg_body(w_hbm, srcm_hbm, dstm_hbm, zvec_hbm, out_hbm,
                 sidx0, sidx1, didx, vals0, vals1, deg_sh, sem0, sem1):
    c = lax.axis_index("c")
    s = lax.axis_index("s")
    w = s * 2 + c
    sidx = (sidx0, sidx1)
    vals = (vals0, vals1)
    sems = (sem0, sem1)
    pltpu.sync_copy(zvec_hbm, deg_sh.at[pl.ds(s * 640, 640)])
    plsc.subcore_barrier()

    def prefetch(k, bi):
        pltpu.sync_copy(srcm_hbm.at[w, k], sidx[bi])
        pltpu.async_copy(w_hbm.at[sidx[bi]], vals[bi], sems[bi])

    def consume(k, bi):
        pltpu.make_async_copy(w_hbm.at[sidx[bi]], vals[bi], sems[bi]).wait()
        pltpu.sync_copy(dstm_hbm.at[w, k], didx)
        pltpu.sync_copy(vals[bi], deg_sh.at[didx], add=True)

    prefetch(0, 0)

    def body(i, _):
        prefetch(2 * i + 1, 1)
        consume(2 * i, 0)

        @pl.when(i < 19)
        def _():
            prefetch(2 * i + 2, 0)
        consume(2 * i + 1, 1)
        return 0

    lax.fori_loop(0, 20, body, 0)
    plsc.subcore_barrier()
    pltpu.sync_copy(deg_sh.at[pl.ds(s * 640, 640)],
                    out_hbm.at[c].at[pl.ds(s * 640, 640)])


@functools.lru_cache(maxsize=None)
def _sc_deg_perm_build():
    return pl.kernel(
        _sc_deg_perm_body,
        out_type=[jax.ShapeDtypeStruct((2, N), f32),
                  jax.ShapeDtypeStruct((N, D), f32),
                  jax.ShapeDtypeStruct((2 * N,), i32)],
        mesh=_sc_mesh(),
        scratch_types=[
            pltpu.VMEM((128,), i32),
            pltpu.VMEM((128,), i32),
            pltpu.VMEM((128,), i32),
            pltpu.VMEM((128,), f32),
            pltpu.VMEM((128,), f32),
            pltpu.VMEM_SHARED((N,), f32),
            pltpu.VMEM((128,), i32),
            pltpu.VMEM((128,), i32),
            pltpu.VMEM((128,), i32),
            pltpu.VMEM((128, D), f32),
            pltpu.SemaphoreType.DMA,
            pltpu.SemaphoreType.DMA,
            pltpu.SemaphoreType.DMA,
        ],
    )


def _sc_deg_perm(w, src_deg, dst_deg, zvec, ht, spos, gpos, ids):
    degp, htp, _ = _sc_deg_perm_build()(w, src_deg, dst_deg, zvec,
                                        ht, spos, gpos, ids)
    return degp, htp


def _sc_deg_perm_body(w_hbm, srcm_hbm, dstm_hbm, zvec_hbm,
                      ht_hbm, spos_hbm, gpos_hbm, ids_hbm,
                      deg_hbm, out_hbm, invb_hbm,
                      sidx0, sidx1, didx, vals0, vals1, deg_sh,
                      sposc, idsc, pc, rows, sem0, sem1, sem):
    c = lax.axis_index("c")
    s = lax.axis_index("s")
    w = s * 2 + c
    sidx = (sidx0, sidx1)
    vals = (vals0, vals1)
    sems = (sem0, sem1)
    NCH = N // 128

    pltpu.sync_copy(zvec_hbm, deg_sh.at[pl.ds(s * 640, 640)])

    def scat(k, _):
        cid = s + 16 * k
        pltpu.sync_copy(spos_hbm.at[pl.ds(cid * 128, 128)], sposc)

        def adj(j, _):
            sposc[pl.ds(j * 16, 16)] = sposc[pl.ds(j * 16, 16)] + c * N
            return 0
        lax.fori_loop(0, 8, adj, 0, unroll=True)
        pltpu.sync_copy(ids_hbm.at[pl.ds(cid * 128, 128)], idsc)
        pltpu.async_copy(idsc, invb_hbm.at[sposc], sem).wait()
        return 0

    lax.fori_loop(0, NCH // 16, scat, 0)
    plsc.subcore_barrier()

    def gath(k, _):
        inner = s + 16 * k

        @pl.when(inner < NCH // 2)
        def _():
            cid = c + 2 * inner
            pltpu.sync_copy(gpos_hbm.at[pl.ds(cid * 128, 128)], pc)

            def adj(j, _):
                pc[pl.ds(j * 16, 16)] = pc[pl.ds(j * 16, 16)] + c * N
                return 0
            lax.fori_loop(0, 8, adj, 0, unroll=True)
            pltpu.async_copy(invb_hbm.at[pc], idsc, sem).wait()

            def clamp(j, _):
                v = idsc[pl.ds(j * 16, 16)]
                v = jnp.minimum(jnp.maximum(v, 0), N - 1)
                idsc[pl.ds(j * 16, 16)] = v
                return 0
            lax.fori_loop(0, 8, clamp, 0, unroll=True)
            pltpu.async_copy(ht_hbm.at[idsc], rows, sem).wait()
            pltpu.sync_copy(rows, out_hbm.at[pl.ds(cid * 128, 128)])
        return 0

    lax.fori_loop(0, 3, gath, 0)

    def prefetch(k, bi):
        pltpu.sync_copy(srcm_hbm.at[w, k], sidx[bi])
        pltpu.async_copy(w_hbm.at[sidx[bi]], vals[bi], sems[bi])

    def consume(k, bi):
        pltpu.make_async_copy(w_hbm.at[sidx[bi]], vals[bi], sems[bi]).wait()
        pltpu.sync_copy(dstm_hbm.at[w, k], didx)
        pltpu.sync_copy(vals[bi], deg_sh.at[didx], add=True)

    prefetch(0, 0)

    def dbody(i, _):
        prefetch(2 * i + 1, 1)
        consume(2 * i, 0)

        @pl.when(i < 19)
        def _():
            prefetch(2 * i + 2, 0)
        consume(2 * i + 1, 1)
        return 0

    lax.fori_loop(0, 20, dbody, 0)
    plsc.subcore_barrier()
    pltpu.sync_copy(deg_sh.at[pl.ds(s * 640, 640)],
                    deg_hbm.at[c].at[pl.ds(s * 640, 640)])


def _full(shape):
    return pl.BlockSpec(shape, lambda *_: tuple(0 for _ in shape))


def _to_col(v_row):
    L = v_row.shape[1]
    eye = jnp.where(lax.broadcasted_iota(i32, (L, L), 0)
                    == lax.broadcasted_iota(i32, (L, L), 1), 1.0, 0.0)
    return jnp.sum(eye * v_row, axis=1, keepdims=True)


def _to_row(v_col):
    L = v_col.shape[0]
    eye = jnp.where(lax.broadcasted_iota(i32, (L, L), 0)
                    == lax.broadcasted_iota(i32, (L, L), 1), 1.0, 0.0)
    return jnp.sum(eye * v_col, axis=0, keepdims=True)


def _matmul_body(x_ref, w_ref, o_ref):
    o_ref[...] = jnp.dot(x_ref[...], w_ref[...], preferred_element_type=f32)


def _tc_matmul(x, w):
    return pl.pallas_call(
        _matmul_body,
        grid=(NB,),
        in_specs=[pl.BlockSpec((BLK, D), lambda b: (b, 0)), _full((D, D))],
        out_specs=pl.BlockSpec((BLK, D), lambda b: (b, 0)),
        out_shape=jax.ShapeDtypeStruct((N, D), f32),
    )(x, w)


def _scale_body(ht_ref, degp_ref, m_ref, g_ref, disc_ref, disi_ref):
    deg = 1.0 + degp_ref[0, :] + degp_ref[1, :]
    disc = lax.rsqrt(deg)
    degi = deg - 1.0
    disi = jnp.where(degi > 0.0, lax.rsqrt(jnp.maximum(degi, 1e-30)), 0.0)
    m = m_ref[...]
    scal = _to_col((disc * m).reshape(1, BLK))
    g = ht_ref[...] * scal
    g_ref[0] = g[:, :128]
    g_ref[1] = g[:, 128:]
    disc_ref[...] = disc
    disi_ref[...] = disi


def _tc_scale(ht, degp, m):
    return pl.pallas_call(
        _scale_body,
        grid=(NB,),
        in_specs=[
            pl.BlockSpec((BLK, D), lambda b: (b, 0)),
            pl.BlockSpec((2, BLK), lambda b: (0, b)),
            pl.BlockSpec((BLK,), lambda b: (b,)),
        ],
        out_specs=[
            pl.BlockSpec((2, BLK, 128), lambda b: (0, b, 0)),
            pl.BlockSpec((BLK,), lambda b: (b,)),
            pl.BlockSpec((BLK,), lambda b: (b,)),
        ],
        out_shape=[
            jax.ShapeDtypeStruct((2, N, 128), f32),
            jax.ShapeDtypeStruct((N,), f32),
            jax.ShapeDtypeStruct((N,), f32),
        ],
    )(ht, degp, m)


def _fin_body(agg_ref, g_ref, disc_ref, disi_ref, m_ref, b_ref,
              h_ref, gi_ref):
    agg = jnp.concatenate([agg_ref[0], agg_ref[1]], axis=1)
    g = jnp.concatenate([g_ref[0], g_ref[1]], axis=1)
    disc_c = _to_col(disc_ref[...].reshape(1, BLK))
    h = jnp.maximum((agg + g) * disc_c + b_ref[...].reshape(1, D), 0.0)
    h_ref[...] = h
    gii_c = _to_col((disi_ref[...] * m_ref[...]).reshape(1, BLK))
    gi = h * gii_c
    gi_ref[0] = gi[:, :128]
    gi_ref[1] = gi[:, 128:]


def _tc_fin(agg, g, disc, disi, m, b):
    return pl.pallas_call(
        _fin_body,
        grid=(NB,),
        in_specs=[
            pl.BlockSpec((2, BLK, 128), lambda b: (0, b, 0)),
            pl.BlockSpec((2, BLK, 128), lambda b: (0, b, 0)),
            pl.BlockSpec((BLK,), lambda b: (b,)),
            pl.BlockSpec((BLK,), lambda b: (b,)),
            pl.BlockSpec((BLK,), lambda b: (b,)),
            _full((D,)),
        ],
        out_specs=[
            pl.BlockSpec((BLK, D), lambda b: (b, 0)),
            pl.BlockSpec((2, BLK, 128), lambda b: (0, b, 0)),
        ],
        out_shape=[
            jax.ShapeDtypeStruct((N, D), f32),
            jax.ShapeDtypeStruct((2, N, 128), f32),
        ],
    )(agg, g, disc, disi, m, b)


def _score_body(h_ref, aggi_ref, disi_ref, sc_ref):
    aggi = jnp.concatenate([aggi_ref[0], aggi_ref[1]], axis=1)
    disi_c = _to_col(disi_ref[...].reshape(1, BLK))
    d = h_ref[...] - aggi * disi_c
    sc_ref[...] = jnp.sum(jnp.abs(d), axis=1, keepdims=True)


def _tc_score(h, aggi, disi):
    return pl.pallas_call(
        _score_body,
        grid=(NB,),
        in_specs=[
            pl.BlockSpec((BLK, D), lambda b: (b, 0)),
            pl.BlockSpec((2, BLK, 128), lambda b: (0, b, 0)),
            pl.BlockSpec((BLK,), lambda b: (b,)),
        ],
        out_specs=pl.BlockSpec((BLK, 1), lambda b: (b, 0)),
        out_shape=jax.ShapeDtypeStruct((N, 1), f32),
    )(h, aggi, disi)


def _scorerank_body(blo_ref, bhi_ref, h_ref, aggi_ref, disi_ref,
                    batc_ref, batf_ref, m_ref, rank_ref,
                    scc_s, scr_s, acc):
    pid = pl.program_id(0)

    @pl.when(pid < NB)
    def _():
        aggi = jnp.concatenate([aggi_ref[0], aggi_ref[1]], axis=1)
        disi_c = _to_col(disi_ref[...].reshape(1, BLK))
        d = h_ref[...] - aggi * disi_c
        sc = jnp.sum(jnp.abs(d), axis=1, keepdims=True)
        scc_s[pl.ds(pid * BLK, BLK), :] = sc
        scr_s[pl.ds(pid * BLK, BLK)] = _to_row(sc).reshape(BLK)

    @pl.when(pid >= NB)
    def _():
        bi = pid - NB
        sci = scc_s[pl.ds(bi * BLK, BLK), :]
        bati = batc_ref[...]
        gi = (lax.broadcasted_iota(i32, (BLK, 1), 0) + bi * BLK)
        acc[...] = jnp.zeros((BLK, 1), f32)

        def jblk(j, _):
            scj = scr_s[pl.ds(j * BLK, BLK)].reshape(1, BLK)
            batj = batf_ref[pl.ds(j * BLK, BLK)].reshape(1, BLK)
            mj = m_ref[pl.ds(j * BLK, BLK)].reshape(1, BLK)
            gj = (lax.broadcasted_iota(i32, (1, BLK), 1) + j * BLK)
            same = (bati == batj) & (mj > 0.0)
            beat = (scj > sci) | ((scj == sci) & (gj < gi))
            acc[...] += jnp.sum(jnp.where(same & beat, 1.0, 0.0),
                                axis=1, keepdims=True)
            return 0

        lax.fori_loop(blo_ref[bi], bhi_ref[bi], jblk, 0)
        rank_ref[...] = acc[...]


def _tc_scorerank(h, aggi, disi, batc, batf, m, blo, bhi):
    bmod = lambda p: (p % NB, 0)
    return pl.pallas_call(
        _scorerank_body,
        grid=(2 * NB,),
        in_specs=[
            pl.BlockSpec(memory_space=pltpu.SMEM),
            pl.BlockSpec(memory_space=pltpu.SMEM),
            pl.BlockSpec((BLK, D), bmod),
            pl.BlockSpec((2, BLK, 128), lambda p: (0, p % NB, 0)),
            pl.BlockSpec((BLK,), lambda p: (p % NB,)),
            pl.BlockSpec((BLK, 1), bmod),
            _full((N,)),
            _full((N,)),
        ],
        out_specs=pl.BlockSpec((BLK, 1), bmod),
        out_shape=jax.ShapeDtypeStruct((N, 1), f32),
        scratch_shapes=[pltpu.VMEM((N, 1), f32), pltpu.VMEM((N,), f32),
                        pltpu.VMEM((BLK, 1), f32)],
    )(blo, bhi, h, aggi, disi, batc, batf, m)


def _pool_body(gs_ref, ge_ref, rank_ref, bat_ref, mold_ref, h_ref,
               mnew_ref, spos_ref, gpos_ref, x_ref, k_ref,
               mx_acc, ms_acc):
    iota_g = lax.broadcasted_iota(i32, (G, 1), 0).astype(f32)

    def blk_oh(b):
        batb = bat_ref[pl.ds(b * BLK, BLK)].reshape(1, BLK)
        return jnp.where(iota_g == batb, 1.0, 0.0)

    def cnt_b(b, cnt):
        mb = mold_ref[pl.ds(b * BLK, BLK)].reshape(1, BLK)
        return cnt + jnp.sum(blk_oh(b) * mb, axis=1, keepdims=True)
    counts = lax.fori_loop(0, NB, cnt_b, jnp.zeros((G, 1), f32))
    k = jnp.ceil(0.5 * counts)
    tri = jnp.where(lax.broadcasted_iota(i32, (G, G), 0)
                    > lax.broadcasted_iota(i32, (G, G), 1), 1.0, 0.0)
    start2 = jnp.sum(tri * _to_row(k), axis=1, keepdims=True)

    tri_le = jnp.where(lax.broadcasted_iota(i32, (BLK, BLK), 0)
                       <= lax.broadcasted_iota(i32, (BLK, BLK), 1), 1.0, 0.0)

    def mk_b(b, carry):
        oh = blk_oh(b)
        mb = mold_ref[pl.ds(b * BLK, BLK)]
        rb = rank_ref[pl.ds(b * BLK, BLK)]
        kb = jnp.sum(oh * k, axis=0)
        s2b = jnp.sum(oh * start2, axis=0)
        mn = jnp.where(rb < kb, mb, 0.0)
        mnew_ref[pl.ds(b * BLK, BLK)] = mn
        dump = (NP + (lax.broadcasted_iota(i32, (1, BLK), 1) + b * BLK)
                % (N - NP)).astype(f32).reshape(BLK)
        sp = jnp.where(mn > 0.0, jnp.minimum(s2b + rb, N - 1.0), dump)
        spos_ref[pl.ds(b * BLK, BLK)] = sp.astype(i32)
        mn_c = _to_col(mn.reshape(1, BLK))
        incl = jnp.sum(tri_le * mn_c, axis=0)
        gp = jnp.where(mn > 0.0, carry + incl - mn, dump)
        gpos_ref[pl.ds(b * BLK, BLK)] = gp.astype(i32)
        return carry + jnp.sum(mn)

    lax.fori_loop(0, NB, mk_b, jnp.zeros((), f32))

    def ms_b(b, _):
        mn = mnew_ref[pl.ds(b * BLK, BLK)].reshape(1, BLK)
        oh = blk_oh(b) * mn
        ms_acc[...] += jnp.dot(oh, h_ref[pl.ds(b * BLK, BLK), :],
                               preferred_element_type=f32)
        return 0
    ms_acc[...] = jnp.zeros((G, D), f32)
    lax.fori_loop(0, NB, ms_b, 0)

    mx_acc[...] = jnp.full((G, D), -jnp.inf, f32)

    def mx_g(g, _):
        b0 = gs_ref[g] // BLK
        b1 = (ge_ref[g] + BLK - 1) // BLK
        gf = g.astype(f32)

        def mb(b, acc):
            base = b * BLK
            batb = bat_ref[pl.ds(base, BLK)].reshape(1, BLK)
            mn = mnew_ref[pl.ds(base, BLK)].reshape(1, BLK)
            sel = jnp.where((batb == gf) & (mn > 0.0), 1.0, 0.0)
            sel_c = _to_col(sel)
            hb = jnp.where(sel_c > 0.0, h_ref[pl.ds(base, BLK), :], -jnp.inf)
            return jnp.maximum(acc, jnp.max(hb, axis=0))

        mxg = lax.fori_loop(b0, b1, mb, jnp.full((D,), -jnp.inf, f32))
        eg = lax.broadcasted_iota(i32, (G, 1), 0) == g
        mx_acc[...] = jnp.where(eg, mxg.reshape(1, D), mx_acc[...])
        return 0

    lax.fori_loop(0, G, mx_g, 0)
    mx = mx_acc[...]
    mx = jnp.where(jnp.isinf(mx), 0.0, mx)
    x_ref[:, :D] = mx
    x_ref[:, D:] = ms_acc[...] / jnp.maximum(k, 1.0)
    k_ref[...] = k


def _tc_pool(rank, bat, mold, h, gs, ge):
    return pl.pallas_call(
        _pool_body,
        in_specs=[
            pl.BlockSpec(memory_space=pltpu.SMEM),
            pl.BlockSpec(memory_space=pltpu.SMEM),
            _full((N,)),
            _full((N,)),
            _full((N,)),
            _full((N, D)),
        ],
        out_specs=[
            _full((N,)), _full((N,)), _full((N,)),
            _full((G, 2 * D)), _full((G, 1)),
        ],
        out_shape=[
            jax.ShapeDtypeStruct((N,), f32),
            jax.ShapeDtypeStruct((N,), i32),
            jax.ShapeDtypeStruct((N,), i32),
            jax.ShapeDtypeStruct((G, 2 * D), f32),
            jax.ShapeDtypeStruct((G, 1), f32),
        ],
        scratch_shapes=[pltpu.VMEM((G, D), f32), pltpu.VMEM((G, D), f32)],
    )(gs, ge, rank, bat, mold, h)


def _readout_body(gs_ref, ge_ref, bat_ref, m_ref, k_ref, h_ref, x_ref,
                  mx_acc, ms_acc):
    iota_g = lax.broadcasted_iota(i32, (G, 1), 0).astype(f32)

    def ms_b(b, _):
        batb = bat_ref[pl.ds(b * BLK, BLK)].reshape(1, BLK)
        mn = m_ref[pl.ds(b * BLK, BLK)].reshape(1, BLK)
        oh = jnp.where(iota_g == batb, 1.0, 0.0) * mn
        ms_acc[...] += jnp.dot(oh, h_ref[pl.ds(b * BLK, BLK), :],
                               preferred_element_type=f32)
        return 0
    ms_acc[...] = jnp.zeros((G, D), f32)
    lax.fori_loop(0, NB, ms_b, 0)

    mx_acc[...] = jnp.full((G, D), -jnp.inf, f32)

    def mx_g(g, _):
        b0 = gs_ref[g] // BLK
        b1 = (ge_ref[g] + BLK - 1) // BLK
        gf = g.astype(f32)

        def mb(b, acc):
            base = b * BLK
            batb = bat_ref[pl.ds(base, BLK)].reshape(1, BLK)
            mn = m_ref[pl.ds(base, BLK)].reshape(1, BLK)
            sel = jnp.where((batb == gf) & (mn > 0.0), 1.0, 0.0)
            sel_c = _to_col(sel)
            hb = jnp.where(sel_c > 0.0, h_ref[pl.ds(base, BLK), :], -jnp.inf)
            return jnp.maximum(acc, jnp.max(hb, axis=0))

        mxg = lax.fori_loop(b0, b1, mb, jnp.full((D,), -jnp.inf, f32))
        eg = lax.broadcasted_iota(i32, (G, 1), 0) == g
        mx_acc[...] = jnp.where(eg, mxg.reshape(1, D), mx_acc[...])
        return 0

    lax.fori_loop(0, G, mx_g, 0)
    mx = mx_acc[...]
    mx = jnp.where(jnp.isinf(mx), 0.0, mx)
    x_ref[:, :D] = mx
    x_ref[:, D:] = ms_acc[...] / jnp.maximum(k_ref[...], 1.0)


def _tc_readout(bat, m, kcnt, h, gs, ge):
    return pl.pallas_call(
        _readout_body,
        in_specs=[
            pl.BlockSpec(memory_space=pltpu.SMEM),
            pl.BlockSpec(memory_space=pltpu.SMEM),
            _full((N,)),
            _full((N,)),
            _full((G, 1)),
            _full((N, D)),
        ],
        out_specs=_full((G, 2 * D)),
        out_shape=jax.ShapeDtypeStruct((G, 2 * D), f32),
        scratch_shapes=[pltpu.VMEM((G, D), f32), pltpu.VMEM((G, D), f32)],
    )(gs, ge, bat, m, kcnt, h)


def _head_body(x1_ref, x2_ref, x3_ref, w1_ref, b1_ref, w2_ref, b2_ref,
               w3_ref, b3_ref, o_ref):
    z = (jnp.maximum(x1_ref[...], 0.0) + jnp.maximum(x2_ref[...], 0.0)
         + jnp.maximum(x3_ref[...], 0.0))
    z = jnp.maximum(jnp.dot(z, w1_ref[...], preferred_element_type=f32)
                    + b1_ref[...][None, :], 0.0)
    z = jnp.maximum(jnp.dot(z, w2_ref[...], preferred_element_type=f32)
                    + b2_ref[...][None, :], 0.0)
    o = jnp.dot(z, w3_ref[...], preferred_element_type=f32) + b3_ref[...][None, :]
    omax = jnp.max(o, axis=1, keepdims=True)
    ex = jnp.exp(o - omax)
    o_ref[...] = o - omax - jnp.log(jnp.sum(ex, axis=1, keepdims=True))


def _tc_head(x1, x2, x3, w1, b1, w2, b2, w3, b3):
    specs = [_full(a.shape) for a in (x1, x2, x3, w1, b1, w2, b2, w3, b3)]
    return pl.pallas_call(
        _head_body,
        in_specs=specs,
        out_specs=_full((G, 16)),
        out_shape=jax.ShapeDtypeStruct((G, 16), f32),
    )(x1, x2, x3, w1, b1, w2, b2, w3, b3)


def kernel(x, edge_index, batch, Wc1, bc1, Wc2, bc2, Wc3, bc3,
           Wl1, bl1, Wl2, bl2, Wl3, bl3):
    n = x.shape[0]
    E = edge_index.shape[1]

    x_p = jnp.zeros((N, D), f32).at[:n].set(x)
    batch_p = jnp.full((N,), G, i32).at[:n].set(batch)
    batf = batch_p.astype(f32)
    batc = batf.reshape(N, 1)
    m0 = (jnp.arange(N) < n).astype(f32)
    npad = EP - E
    pad_rows = n + (jnp.arange(npad, dtype=i32) % (N - n))
    src_p = jnp.concatenate([edge_index[0], pad_rows])
    dst_p = jnp.concatenate([edge_index[1], pad_rows])
    src_spmm = src_p.reshape(16, 80, 128)
    dst_spmm = dst_p.reshape(16, 80, 128)
    src_deg = src_p.reshape(32, 40, 128)
    dst_deg = dst_p.reshape(32, 40, 128)
    zrows = jnp.zeros((640, 128), f32)
    zvec = jnp.zeros((640,), f32)
    ids_p = jnp.arange(N, dtype=i32)
    bounds = jnp.arange(0, N + 1, BLK)
    bstart = batch_p[jnp.minimum(bounds[:-1], N - 1)]
    bend = batch_p[jnp.minimum(bounds[1:] - 1, N - 1)]
    gsearch = jnp.searchsorted(batch_p[:n], jnp.arange(G + 1, dtype=i32)).astype(i32)
    gs, ge = gsearch[:-1], gsearch[1:]
    blo = jnp.searchsorted(bend, bstart, side="left").astype(i32)
    bhi = jnp.searchsorted(bstart, bend, side="right").astype(i32)

    def conv(h_in, W, b, m, spos, gpos):
        ht = _tc_matmul(h_in, W)
        if spos is not None:
            degp, ht = _sc_deg_perm(m, src_deg, dst_deg, zvec,
                                    ht, spos, gpos, ids_p)
        else:
            degp = _sc_deg(m, src_deg, dst_deg, zvec)
        g, disc, disi = _tc_scale(ht, degp, m)
        agg = _sc_spmm(g.reshape(2 * N, 128), src_spmm, dst_spmm, zrows)
        h, gi = _tc_fin(agg, g, disc, disi, m, b)
        return h, gi, disi

    def score_pool(h, gi, disi, m):
        aggi = _sc_spmm(gi.reshape(2 * N, 128), src_spmm, dst_spmm, zrows)
        rank = _tc_scorerank(h, aggi, disi, batc, batf, m, blo, bhi)
        return _tc_pool(rank.reshape(N), batf, m, h, gs, ge)

    h1, gi1, disi1 = conv(x_p, Wc1, bc1, m0, None, None)
    m1, spos1, gpos1, x1, k1 = score_pool(h1, gi1, disi1, m0)
    h2, gi2, disi2 = conv(h1, Wc2, bc2, m1, spos1, gpos1)
    m2, spos2, gpos2, x2, k2 = score_pool(h2, gi2, disi2, m1)
    h3, _, _ = conv(h2, Wc3, bc3, m2, spos2, gpos2)
    x3 = _tc_readout(batf, m2, k2, h3, gs, ge)
    return _tc_head(x1, x2, x3, Wl1, bl1, Wl2, bl2, Wl3, bl3)


_ = math


# speedup vs baseline: 1.0170x; 1.0170x over previous
import functools
import math

import jax
import jax.numpy as jnp
from jax import lax
from jax.experimental import pallas as pl
from jax.experimental.pallas import tpu as pltpu
from jax.experimental.pallas import tpu_sc as plsc

N = 10240
NP = 10000
D = 256
G = 64
EP = 163840
NB = 40
BLK = 256

f32 = jnp.float32
i32 = jnp.int32

def _sc_mesh():
    return plsc.VectorSubcoreMesh(core_axis_name="c", subcore_axis_name="s",
                                  num_cores=2, num_subcores=16)


@functools.lru_cache(maxsize=None)
def _sc_spmm_build():
    return pl.kernel(
        _sc_spmm_body,
        out_type=jax.ShapeDtypeStruct((2, N, 128), f32),
        mesh=_sc_mesh(),
        scratch_types=[
            pltpu.VMEM((128,), i32),
            pltpu.VMEM((128,), i32),
            pltpu.VMEM((128,), i32),
            pltpu.VMEM((128, 128), f32),
            pltpu.VMEM((128, 128), f32),
            pltpu.VMEM_SHARED((N, 128), f32),
            pltpu.SemaphoreType.DMA,
            pltpu.SemaphoreType.DMA,
        ],
    )


def _sc_spmm(g_flat, src_spmm, dst_spmm, zrows):
    return _sc_spmm_build()(g_flat, src_spmm, dst_spmm, zrows)


def _sc_spmm_body(g_hbm, srcm_hbm, dstm_hbm, zrows_hbm, out_hbm,
                  sidx0, sidx1, didx, rows0, rows1, acc_sh, sem0, sem1):
    c = lax.axis_index("c")
    s = lax.axis_index("s")
    sidx = (sidx0, sidx1)
    rows = (rows0, rows1)
    sems = (sem0, sem1)
    pltpu.sync_copy(zrows_hbm, acc_sh.at[pl.ds(s * 640, 640)])
    plsc.subcore_barrier()

    def prefetch(k, bi):
        pltpu.sync_copy(srcm_hbm.at[s, k], sidx[bi])

        def adj(j, _):
            sidx[bi][pl.ds(j * 16, 16)] = sidx[bi][pl.ds(j * 16, 16)] + c * N
            return 0
        lax.fori_loop(0, 8, adj, 0, unroll=True)
        pltpu.async_copy(g_hbm.at[sidx[bi]], rows[bi], sems[bi])

    def consume(k, bi):
        pltpu.make_async_copy(g_hbm.at[sidx[bi]], rows[bi], sems[bi]).wait()
        pltpu.sync_copy(dstm_hbm.at[s, k], didx)
        pltpu.sync_copy(rows[bi], acc_sh.at[didx], add=True)

    prefetch(0, 0)

    def body(i, _):
        prefetch(2 * i + 1, 1)
        consume(2 * i, 0)

        @pl.when(i < 39)
        def _():
            prefetch(2 * i + 2, 0)
        consume(2 * i + 1, 1)
        return 0

    lax.fori_loop(0, 40, body, 0)
    plsc.subcore_barrier()
    pltpu.sync_copy(acc_sh.at[pl.ds(s * 640, 640)],
                    out_hbm.at[c].at[pl.ds(s * 640, 640)])


@functools.lru_cache(maxsize=None)
def _sc_deg_build():
    return pl.kernel(
        _sc_deg_body,
        out_type=jax.ShapeDtypeStruct((2, N), f32),
        mesh=_sc_mesh(),
        scratch_types=[
            pltpu.VMEM((128,), i32),
            pltpu.VMEM((128,), i32),
            pltpu.VMEM((128,), i32),
            pltpu.VMEM((128,), f32),
            pltpu.VMEM((128,), f32),
            pltpu.VMEM_SHARED((N,), f32),
            pltpu.SemaphoreType.DMA,
            pltpu.SemaphoreType.DMA,
        ],
    )


def _sc_deg(w, src_deg, dst_deg, zvec):
    return _sc_deg_build()(w, src_deg, dst_deg, zvec)


def _sc_deg_body(w_hbm, srcm_hbm, dstm_hbm, zvec_hbm, out_hbm,
                 sidx0, sidx1, didx, vals0, vals1, deg_sh, sem0, sem1):
    c = lax.axis_index("c")
    s = lax.axis_index("s")
    w = s * 2 + c
    sidx = (sidx0, sidx1)
    vals = (vals0, vals1)
    sems = (sem0, sem1)
    pltpu.sync_copy(zvec_hbm, deg_sh.at[pl.ds(s * 640, 640)])
    plsc.subcore_barrier()

    def prefetch(k, bi):
        pltpu.sync_copy(srcm_hbm.at[w, k], sidx[bi])
        pltpu.async_copy(w_hbm.at[sidx[bi]], vals[bi], sems[bi])

    def consume(k, bi):
        pltpu.make_async_copy(w_hbm.at[sidx[bi]], vals[bi], sems[bi]).wait()
        pltpu.sync_copy(dstm_hbm.at[w, k], didx)
        pltpu.sync_copy(vals[bi], deg_sh.at[didx], add=True)

    prefetch(0, 0)

    def body(i, _):
        prefetch(2 * i + 1, 1)
        consume(2 * i, 0)

        @pl.when(i < 19)
        def _():
            prefetch(2 * i + 2, 0)
        consume(2 * i + 1, 1)
        return 0

    lax.fori_loop(0, 20, body, 0)
    plsc.subcore_barrier()
    pltpu.sync_copy(deg_sh.at[pl.ds(s * 640, 640)],
                    out_hbm.at[c].at[pl.ds(s * 640, 640)])


@functools.lru_cache(maxsize=None)
def _sc_deg_perm_build():
    return pl.kernel(
        _sc_deg_perm_body,
        out_type=[jax.ShapeDtypeStruct((2, N), f32),
                  jax.ShapeDtypeStruct((N, D), f32),
                  jax.ShapeDtypeStruct((2 * N,), i32)],
        mesh=_sc_mesh(),
        scratch_types=[
            pltpu.VMEM((128,), i32),
            pltpu.VMEM((128,), i32),
            pltpu.VMEM((128,), i32),
            pltpu.VMEM((128,), f32),
            pltpu.VMEM((128,), f32),
            pltpu.VMEM_SHARED((N,), f32),
            pltpu.VMEM((128,), i32),
            pltpu.VMEM((128,), i32),
            pltpu.VMEM((128,), i32),
            pltpu.VMEM((128, D), f32),
            pltpu.SemaphoreType.DMA,
            pltpu.SemaphoreType.DMA,
            pltpu.SemaphoreType.DMA,
        ],
    )


def _sc_deg_perm(w, src_deg, dst_deg, zvec, ht, spos, gpos, ids):
    degp, htp, _ = _sc_deg_perm_build()(w, src_deg, dst_deg, zvec,
                                        ht, spos, gpos, ids)
    return degp, htp


def _sc_deg_perm_body(w_hbm, srcm_hbm, dstm_hbm, zvec_hbm,
                      ht_hbm, spos_hbm, gpos_hbm, ids_hbm,
                      deg_hbm, out_hbm, invb_hbm,
                      sidx0, sidx1, didx, vals0, vals1, deg_sh,
                      sposc, idsc, pc, rows, sem0, sem1, sem):
    c = lax.axis_index("c")
    s = lax.axis_index("s")
    w = s * 2 + c
    sidx = (sidx0, sidx1)
    vals = (vals0, vals1)
    sems = (sem0, sem1)
    NCH = N // 128

    pltpu.sync_copy(zvec_hbm, deg_sh.at[pl.ds(s * 640, 640)])

    def scat(k, _):
        cid = s + 16 * k
        pltpu.sync_copy(spos_hbm.at[pl.ds(cid * 128, 128)], sposc)

        def adj(j, _):
            sposc[pl.ds(j * 16, 16)] = sposc[pl.ds(j * 16, 16)] + c * N
            return 0
        lax.fori_loop(0, 8, adj, 0, unroll=True)
        pltpu.sync_copy(ids_hbm.at[pl.ds(cid * 128, 128)], idsc)
        pltpu.async_copy(idsc, invb_hbm.at[sposc], sem).wait()
        return 0

    lax.fori_loop(0, NCH // 16, scat, 0)
    plsc.subcore_barrier()

    def gath(k, _):
        inner = s + 16 * k

        @pl.when(inner < NCH // 2)
        def _():
            cid = c + 2 * inner
            pltpu.sync_copy(gpos_hbm.at[pl.ds(cid * 128, 128)], pc)

            def adj(j, _):
                pc[pl.ds(j * 16, 16)] = pc[pl.ds(j * 16, 16)] + c * N
                return 0
            lax.fori_loop(0, 8, adj, 0, unroll=True)
            pltpu.async_copy(invb_hbm.at[pc], idsc, sem).wait()

            def clamp(j, _):
                v = idsc[pl.ds(j * 16, 16)]
                v = jnp.minimum(jnp.maximum(v, 0), N - 1)
                idsc[pl.ds(j * 16, 16)] = v
                return 0
            lax.fori_loop(0, 8, clamp, 0, unroll=True)
            pltpu.async_copy(ht_hbm.at[idsc], rows, sem).wait()
            pltpu.sync_copy(rows, out_hbm.at[pl.ds(cid * 128, 128)])
        return 0

    lax.fori_loop(0, 3, gath, 0)

    def prefetch(k, bi):
        pltpu.sync_copy(srcm_hbm.at[w, k], sidx[bi])
        pltpu.async_copy(w_hbm.at[sidx[bi]], vals[bi], sems[bi])

    def consume(k, bi):
        pltpu.make_async_copy(w_hbm.at[sidx[bi]], vals[bi], sems[bi]).wait()
        pltpu.sync_copy(dstm_hbm.at[w, k], didx)
        pltpu.sync_copy(vals[bi], deg_sh.at[didx], add=True)

    prefetch(0, 0)

    def dbody(i, _):
        prefetch(2 * i + 1, 1)
        consume(2 * i, 0)

        @pl.when(i < 19)
        def _():
            prefetch(2 * i + 2, 0)
        consume(2 * i + 1, 1)
        return 0

    lax.fori_loop(0, 20, dbody, 0)
    plsc.subcore_barrier()
    pltpu.sync_copy(deg_sh.at[pl.ds(s * 640, 640)],
                    deg_hbm.at[c].at[pl.ds(s * 640, 640)])


def _full(shape):
    return pl.BlockSpec(shape, lambda *_: tuple(0 for _ in shape))


def _to_col(v_row):
    L = v_row.shape[1]
    eye = jnp.where(lax.broadcasted_iota(i32, (L, L), 0)
                    == lax.broadcasted_iota(i32, (L, L), 1), 1.0, 0.0)
    return jnp.sum(eye * v_row, axis=1, keepdims=True)


def _to_row(v_col):
    L = v_col.shape[0]
    eye = jnp.where(lax.broadcasted_iota(i32, (L, L), 0)
                    == lax.broadcasted_iota(i32, (L, L), 1), 1.0, 0.0)
    return jnp.sum(eye * v_col, axis=0, keepdims=True)


def _matmul_body(x_ref, w_ref, o_ref):
    o_ref[...] = jnp.dot(x_ref[...], w_ref[...], preferred_element_type=f32)


def _tc_matmul(x, w):
    return pl.pallas_call(
        _matmul_body,
        grid=(NB,),
        in_specs=[pl.BlockSpec((BLK, D), lambda b: (b, 0)), _full((D, D))],
        out_specs=pl.BlockSpec((BLK, D), lambda b: (b, 0)),
        out_shape=jax.ShapeDtypeStruct((N, D), f32),
    )(x, w)


def _scale_body(ht_ref, degp_ref, m_ref, g_ref, disc_ref, disi_ref):
    deg = 1.0 + degp_ref[0, :] + degp_ref[1, :]
    disc = lax.rsqrt(deg)
    degi = deg - 1.0
    disi = jnp.where(degi > 0.0, lax.rsqrt(jnp.maximum(degi, 1e-30)), 0.0)
    m = m_ref[...]
    scal = _to_col((disc * m).reshape(1, BLK))
    g = ht_ref[...] * scal
    g_ref[0] = g[:, :128]
    g_ref[1] = g[:, 128:]
    disc_ref[...] = disc
    disi_ref[...] = disi


def _tc_scale(ht, degp, m):
    return pl.pallas_call(
        _scale_body,
        grid=(NB,),
        in_specs=[
            pl.BlockSpec((BLK, D), lambda b: (b, 0)),
            pl.BlockSpec((2, BLK), lambda b: (0, b)),
            pl.BlockSpec((BLK,), lambda b: (b,)),
        ],
        out_specs=[
            pl.BlockSpec((2, BLK, 128), lambda b: (0, b, 0)),
            pl.BlockSpec((BLK,), lambda b: (b,)),
            pl.BlockSpec((BLK,), lambda b: (b,)),
        ],
        out_shape=[
            jax.ShapeDtypeStruct((2, N, 128), f32),
            jax.ShapeDtypeStruct((N,), f32),
            jax.ShapeDtypeStruct((N,), f32),
        ],
    )(ht, degp, m)


def _fin_body(agg_ref, g_ref, disc_ref, disi_ref, m_ref, b_ref,
              h_ref, gi_ref):
    agg = jnp.concatenate([agg_ref[0], agg_ref[1]], axis=1)
    g = jnp.concatenate([g_ref[0], g_ref[1]], axis=1)
    disc_c = _to_col(disc_ref[...].reshape(1, BLK))
    h = jnp.maximum((agg + g) * disc_c + b_ref[...].reshape(1, D), 0.0)
    h_ref[...] = h
    gii_c = _to_col((disi_ref[...] * m_ref[...]).reshape(1, BLK))
    gi = h * gii_c
    gi_ref[0] = gi[:, :128]
    gi_ref[1] = gi[:, 128:]


def _tc_fin(agg, g, disc, disi, m, b):
    return pl.pallas_call(
        _fin_body,
        grid=(NB,),
        in_specs=[
            pl.BlockSpec((2, BLK, 128), lambda b: (0, b, 0)),
            pl.BlockSpec((2, BLK, 128), lambda b: (0, b, 0)),
            pl.BlockSpec((BLK,), lambda b: (b,)),
            pl.BlockSpec((BLK,), lambda b: (b,)),
            pl.BlockSpec((BLK,), lambda b: (b,)),
            _full((D,)),
        ],
        out_specs=[
            pl.BlockSpec((BLK, D), lambda b: (b, 0)),
            pl.BlockSpec((2, BLK, 128), lambda b: (0, b, 0)),
        ],
        out_shape=[
            jax.ShapeDtypeStruct((N, D), f32),
            jax.ShapeDtypeStruct((2, N, 128), f32),
        ],
    )(agg, g, disc, disi, m, b)


def _score_body(h_ref, aggi_ref, disi_ref, sc_ref):
    aggi = jnp.concatenate([aggi_ref[0], aggi_ref[1]], axis=1)
    disi_c = _to_col(disi_ref[...].reshape(1, BLK))
    d = h_ref[...] - aggi * disi_c
    sc_ref[...] = jnp.sum(jnp.abs(d), axis=1, keepdims=True)


def _tc_score(h, aggi, disi):
    return pl.pallas_call(
        _score_body,
        grid=(NB,),
        in_specs=[
            pl.BlockSpec((BLK, D), lambda b: (b, 0)),
            pl.BlockSpec((2, BLK, 128), lambda b: (0, b, 0)),
            pl.BlockSpec((BLK,), lambda b: (b,)),
        ],
        out_specs=pl.BlockSpec((BLK, 1), lambda b: (b, 0)),
        out_shape=jax.ShapeDtypeStruct((N, 1), f32),
    )(h, aggi, disi)


def _rank_body(blo_ref, bhi_ref, scc_ref, scf_ref, batc_ref, batf_ref,
               m_ref, rank_ref, acc):
    bi = pl.program_id(0)
    sci = scc_ref[pl.ds(bi * BLK, BLK), :]
    bati = batc_ref[pl.ds(bi * BLK, BLK), :]
    gi = (lax.broadcasted_iota(i32, (BLK, 1), 0) + bi * BLK)
    acc[...] = jnp.zeros((BLK, 1), f32)
    lo = blo_ref[bi]
    hi = bhi_ref[bi]

    def jblk(j, _):
        scj = scf_ref[pl.ds(j * BLK, BLK)].reshape(1, BLK)
        batj = batf_ref[pl.ds(j * BLK, BLK)].reshape(1, BLK)
        mj = m_ref[pl.ds(j * BLK, BLK)].reshape(1, BLK)
        gj = (lax.broadcasted_iota(i32, (1, BLK), 1) + j * BLK)
        same = (bati == batj) & (mj > 0.0)
        beat = (scj > sci) | ((scj == sci) & (gj < gi))
        acc[...] += jnp.sum(jnp.where(same & beat, 1.0, 0.0),
                            axis=1, keepdims=True)
        return 0

    lax.fori_loop(lo, hi, jblk, 0)
    rank_ref[...] = acc[...]


def _tc_rank(scc, scf, batc, batf, m, blo, bhi):
    return pl.pallas_call(
        _rank_body,
        grid=(NB,),
        in_specs=[
            pl.BlockSpec(memory_space=pltpu.SMEM),
            pl.BlockSpec(memory_space=pltpu.SMEM),
            _full((N, 1)),
            _full((N,)),
            _full((N, 1)),
            _full((N,)),
            _full((N,)),
        ],
        out_specs=pl.BlockSpec((BLK, 1), lambda b: (b, 0)),
        out_shape=jax.ShapeDtypeStruct((N, 1), f32),
        scratch_shapes=[pltpu.VMEM((BLK, 1), f32)],
    )(blo, bhi, scc, scf, batc, batf, m)


def _pool_body(gs_ref, ge_ref, rank_ref, bat_ref, mold_ref, h_ref,
               mnew_ref, spos_ref, gpos_ref, x_ref, k_ref,
               mx_acc, ms_acc):
    iota_g = lax.broadcasted_iota(i32, (G, 1), 0).astype(f32)

    def blk_oh(b):
        batb = bat_ref[pl.ds(b * BLK, BLK)].reshape(1, BLK)
        return jnp.where(iota_g == batb, 1.0, 0.0)

    def cnt_b(b, cnt):
        mb = mold_ref[pl.ds(b * BLK, BLK)].reshape(1, BLK)
        return cnt + jnp.sum(blk_oh(b) * mb, axis=1, keepdims=True)
    counts = lax.fori_loop(0, NB, cnt_b, jnp.zeros((G, 1), f32))
    k = jnp.ceil(0.5 * counts)
    tri = jnp.where(lax.broadcasted_iota(i32, (G, G), 0)
                    > lax.broadcasted_iota(i32, (G, G), 1), 1.0, 0.0)
    start2 = jnp.sum(tri * _to_row(k), axis=1, keepdims=True)

    tri_le = jnp.where(lax.broadcasted_iota(i32, (BLK, BLK), 0)
                       <= lax.broadcasted_iota(i32, (BLK, BLK), 1), 1.0, 0.0)

    def mk_b(b, carry):
        oh = blk_oh(b)
        mb = mold_ref[pl.ds(b * BLK, BLK)]
        rb = rank_ref[pl.ds(b * BLK, BLK)]
        kb = jnp.sum(oh * k, axis=0)
        s2b = jnp.sum(oh * start2, axis=0)
        mn = jnp.where(rb < kb, mb, 0.0)
        mnew_ref[pl.ds(b * BLK, BLK)] = mn
        dump = (NP + (lax.broadcasted_iota(i32, (1, BLK), 1) + b * BLK)
                % (N - NP)).astype(f32).reshape(BLK)
        sp = jnp.where(mn > 0.0, jnp.minimum(s2b + rb, N - 1.0), dump)
        spos_ref[pl.ds(b * BLK, BLK)] = sp.astype(i32)
        mn_c = _to_col(mn.reshape(1, BLK))
        incl = jnp.sum(tri_le * mn_c, axis=0)
        gp = jnp.where(mn > 0.0, carry + incl - mn, dump)
        gpos_ref[pl.ds(b * BLK, BLK)] = gp.astype(i32)
        return carry + jnp.sum(mn)

    lax.fori_loop(0, NB, mk_b, jnp.zeros((), f32))

    def ms_b(b, _):
        mn = mnew_ref[pl.ds(b * BLK, BLK)].reshape(1, BLK)
        oh = blk_oh(b) * mn
        ms_acc[...] += jnp.dot(oh, h_ref[pl.ds(b * BLK, BLK), :],
                               preferred_element_type=f32)
        return 0
    ms_acc[...] = jnp.zeros((G, D), f32)
    lax.fori_loop(0, NB, ms_b, 0)

    mx_acc[...] = jnp.full((G, D), -jnp.inf, f32)

    def mx_g(g, _):
        b0 = gs_ref[g] // BLK
        b1 = (ge_ref[g] + BLK - 1) // BLK
        gf = g.astype(f32)

        def mb(b, acc):
            base = b * BLK
            batb = bat_ref[pl.ds(base, BLK)].reshape(1, BLK)
            mn = mnew_ref[pl.ds(base, BLK)].reshape(1, BLK)
            sel = jnp.where((batb == gf) & (mn > 0.0), 1.0, 0.0)
            sel_c = _to_col(sel)
            hb = jnp.where(sel_c > 0.0, h_ref[pl.ds(base, BLK), :], -jnp.inf)
            return jnp.maximum(acc, jnp.max(hb, axis=0))

        mxg = lax.fori_loop(b0, b1, mb, jnp.full((D,), -jnp.inf, f32))
        eg = lax.broadcasted_iota(i32, (G, 1), 0) == g
        mx_acc[...] = jnp.where(eg, mxg.reshape(1, D), mx_acc[...])
        return 0

    lax.fori_loop(0, G, mx_g, 0)
    mx = mx_acc[...]
    mx = jnp.where(jnp.isinf(mx), 0.0, mx)
    x_ref[:, :D] = mx
    x_ref[:, D:] = ms_acc[...] / jnp.maximum(k, 1.0)
    k_ref[...] = k


def _tc_pool(rank, bat, mold, h, gs, ge):
    return pl.pallas_call(
        _pool_body,
        in_specs=[
            pl.BlockSpec(memory_space=pltpu.SMEM),
            pl.BlockSpec(memory_space=pltpu.SMEM),
            _full((N,)),
            _full((N,)),
            _full((N,)),
            _full((N, D)),
        ],
        out_specs=[
            _full((N,)), _full((N,)), _full((N,)),
            _full((G, 2 * D)), _full((G, 1)),
        ],
        out_shape=[
            jax.ShapeDtypeStruct((N,), f32),
            jax.ShapeDtypeStruct((N,), i32),
            jax.ShapeDtypeStruct((N,), i32),
            jax.ShapeDtypeStruct((G, 2 * D), f32),
            jax.ShapeDtypeStruct((G, 1), f32),
        ],
        scratch_shapes=[pltpu.VMEM((G, D), f32), pltpu.VMEM((G, D), f32)],
    )(gs, ge, rank, bat, mold, h)


def _readout_body(gs_ref, ge_ref, bat_ref, m_ref, k_ref, h_ref, x_ref,
                  mx_acc, ms_acc):
    iota_g = lax.broadcasted_iota(i32, (G, 1), 0).astype(f32)

    def ms_b(b, _):
        batb = bat_ref[pl.ds(b * BLK, BLK)].reshape(1, BLK)
        mn = m_ref[pl.ds(b * BLK, BLK)].reshape(1, BLK)
        oh = jnp.where(iota_g == batb, 1.0, 0.0) * mn
        ms_acc[...] += jnp.dot(oh, h_ref[pl.ds(b * BLK, BLK), :],
                               preferred_element_type=f32)
        return 0
    ms_acc[...] = jnp.zeros((G, D), f32)
    lax.fori_loop(0, NB, ms_b, 0)

    mx_acc[...] = jnp.full((G, D), -jnp.inf, f32)

    def mx_g(g, _):
        b0 = gs_ref[g] // BLK
        b1 = (ge_ref[g] + BLK - 1) // BLK
        gf = g.astype(f32)

        def mb(b, acc):
            base = b * BLK
            batb = bat_ref[pl.ds(base, BLK)].reshape(1, BLK)
            mn = m_ref[pl.ds(base, BLK)].reshape(1, BLK)
            sel = jnp.where((batb == gf) & (mn > 0.0), 1.0, 0.0)
            sel_c = _to_col(sel)
            hb = jnp.where(sel_c > 0.0, h_ref[pl.ds(base, BLK), :], -jnp.inf)
            return jnp.maximum(acc, jnp.max(hb, axis=0))

        mxg = lax.fori_loop(b0, b1, mb, jnp.full((D,), -jnp.inf, f32))
        eg = lax.broadcasted_iota(i32, (G, 1), 0) == g
        mx_acc[...] = jnp.where(eg, mxg.reshape(1, D), mx_acc[...])
        return 0

    lax.fori_loop(0, G, mx_g, 0)
    mx = mx_acc[...]
    mx = jnp.where(jnp.isinf(mx), 0.0, mx)
    x_ref[:, :D] = mx
    x_ref[:, D:] = ms_acc[...] / jnp.maximum(k_ref[...], 1.0)


def _tc_readout(bat, m, kcnt, h, gs, ge):
    return pl.pallas_call(
        _readout_body,
        in_specs=[
            pl.BlockSpec(memory_space=pltpu.SMEM),
            pl.BlockSpec(memory_space=pltpu.SMEM),
            _full((N,)),
            _full((N,)),
            _full((G, 1)),
            _full((N, D)),
        ],
        out_specs=_full((G, 2 * D)),
        out_shape=jax.ShapeDtypeStruct((G, 2 * D), f32),
        scratch_shapes=[pltpu.VMEM((G, D), f32), pltpu.VMEM((G, D), f32)],
    )(gs, ge, bat, m, kcnt, h)


def _head_body(x1_ref, x2_ref, x3_ref, w1_ref, b1_ref, w2_ref, b2_ref,
               w3_ref, b3_ref, o_ref):
    z = (jnp.maximum(x1_ref[...], 0.0) + jnp.maximum(x2_ref[...], 0.0)
         + jnp.maximum(x3_ref[...], 0.0))
    z = jnp.maximum(jnp.dot(z, w1_ref[...], preferred_element_type=f32)
                    + b1_ref[...][None, :], 0.0)
    z = jnp.maximum(jnp.dot(z, w2_ref[...], preferred_element_type=f32)
                    + b2_ref[...][None, :], 0.0)
    o = jnp.dot(z, w3_ref[...], preferred_element_type=f32) + b3_ref[...][None, :]
    omax = jnp.max(o, axis=1, keepdims=True)
    ex = jnp.exp(o - omax)
    o_ref[...] = o - omax - jnp.log(jnp.sum(ex, axis=1, keepdims=True))


def _tc_head(x1, x2, x3, w1, b1, w2, b2, w3, b3):
    specs = [_full(a.shape) for a in (x1, x2, x3, w1, b1, w2, b2, w3, b3)]
    return pl.pallas_call(
        _head_body,
        in_specs=specs,
        out_specs=_full((G, 16)),
        out_shape=jax.ShapeDtypeStruct((G, 16), f32),
    )(x1, x2, x3, w1, b1, w2, b2, w3, b3)


def kernel(x, edge_index, batch, Wc1, bc1, Wc2, bc2, Wc3, bc3,
           Wl1, bl1, Wl2, bl2, Wl3, bl3):
    n = x.shape[0]
    E = edge_index.shape[1]

    x_p = jnp.zeros((N, D), f32).at[:n].set(x)
    batch_p = jnp.full((N,), G, i32).at[:n].set(batch)
    batf = batch_p.astype(f32)
    batc = batf.reshape(N, 1)
    m0 = (jnp.arange(N) < n).astype(f32)
    npad = EP - E
    pad_rows = n + (jnp.arange(npad, dtype=i32) % (N - n))
    src_p = jnp.concatenate([edge_index[0], pad_rows])
    dst_p = jnp.concatenate([edge_index[1], pad_rows])
    src_spmm = src_p.reshape(16, 80, 128)
    dst_spmm = dst_p.reshape(16, 80, 128)
    src_deg = src_p.reshape(32, 40, 128)
    dst_deg = dst_p.reshape(32, 40, 128)
    zrows = jnp.zeros((640, 128), f32)
    zvec = jnp.zeros((640,), f32)
    ids_p = jnp.arange(N, dtype=i32)
    bounds = jnp.arange(0, N + 1, BLK)
    bstart = batch_p[jnp.minimum(bounds[:-1], N - 1)]
    bend = batch_p[jnp.minimum(bounds[1:] - 1, N - 1)]
    gsearch = jnp.searchsorted(batch_p[:n], jnp.arange(G + 1, dtype=i32)).astype(i32)
    gs, ge = gsearch[:-1], gsearch[1:]
    blo = jnp.searchsorted(bend, bstart, side="left").astype(i32)
    bhi = jnp.searchsorted(bstart, bend, side="right").astype(i32)

    def conv(h_in, W, b, m, spos, gpos):
        ht = _tc_matmul(h_in, W)
        if spos is not None:
            degp, ht = _sc_deg_perm(m, src_deg, dst_deg, zvec,
                                    ht, spos, gpos, ids_p)
        else:
            degp = _sc_deg(m, src_deg, dst_deg, zvec)
        g, disc, disi = _tc_scale(ht, degp, m)
        agg = _sc_spmm(g.reshape(2 * N, 128), src_spmm, dst_spmm, zrows)
        h, gi = _tc_fin(agg, g, disc, disi, m, b)
        return h, gi, disi

    def score_pool(h, gi, disi, m):
        aggi = _sc_spmm(gi.reshape(2 * N, 128), src_spmm, dst_spmm, zrows)
        scc = _tc_score(h, aggi, disi)
        rank = _tc_rank(scc, scc.reshape(N), batc, batf, m, blo, bhi)
        return _tc_pool(rank.reshape(N), batf, m, h, gs, ge)

    h1, gi1, disi1 = conv(x_p, Wc1, bc1, m0, None, None)
    m1, spos1, gpos1, x1, k1 = score_pool(h1, gi1, disi1, m0)
    h2, gi2, disi2 = conv(h1, Wc2, bc2, m1, spos1, gpos1)
    m2, spos2, gpos2, x2, k2 = score_pool(h2, gi2, disi2, m1)
    h3, _, _ = conv(h2, Wc3, bc3, m2, spos2, gpos2)
    x3 = _tc_readout(batf, m2, k2, h3, gs, ge)
    return _tc_head(x1, x2, x3, Wl1, bl1, Wl2, bl2, Wl3, bl3)


_ = math
